# baseline (speedup 1.0000x reference)
import functools

import jax
import jax.numpy as jnp
from jax import lax
from jax.experimental import pallas as pl
from jax.experimental.pallas import tpu as pltpu
from jax.experimental.pallas import tpu_sc as plsc

B, T, V, D = 4, 200, 8192, 1024
PPW = 8
ACTIVE = T // PPW
RPW = B * PPW
L = 16

_mesh = plsc.VectorSubcoreMesh(core_axis_name="c", subcore_axis_name="s")


@functools.partial(
    pl.kernel,
    mesh=_mesh,
    out_type=jax.ShapeDtypeStruct((B, T * D), jnp.float32),
    scratch_types=[
        pltpu.VMEM((B, T), jnp.int32),
        pltpu.VMEM((RPW,), jnp.int32),
        pltpu.VMEM((RPW, D), jnp.float32),
        pltpu.SemaphoreType.DMA,
        pltpu.SemaphoreType.DMA,
    ],
)
def _sc_gather(table_hbm, tok_hbm, out_hbm, tok_v, idx_v, rows_v, gsem, wsem):
    wid = lax.axis_index("s") * 2 + lax.axis_index("c")

    @pl.when(wid < ACTIVE)
    def _():
        t0 = wid * PPW
        pltpu.sync_copy(tok_hbm, tok_v)
        t0a = (wid // 2) * L
        r = (wid % 2) * PPW
        vb = [tok_v[b, pl.ds(t0a, L)] for b in range(B)]
        lane = lax.iota(jnp.int32, L)
        bsel = lax.rem(lane, B)
        for half in range(RPW // L):
            pos = r + (half * L) // B + lax.div(lane, B)
            dnums = lax.GatherDimensionNumbers(
                offset_dims=(), collapsed_slice_dims=(0,), start_index_map=(0,)
            )
            spread = [
                lax.gather(
                    v,
                    pos[:, None],
                    dnums,
                    (1,),
                    mode=lax.GatherScatterMode.PROMISE_IN_BOUNDS,
                )
                for v in vb
            ]
            mix = spread[B - 1]
            for b in range(B - 2, -1, -1):
                mix = jnp.where(bsel == b, spread[b], mix)
            idx_v[pl.ds(half * L, L)] = mix + bsel * V
        pltpu.async_copy(table_hbm.at[idx_v], rows_v, gsem).wait()
        for j in range(PPW):
            pltpu.async_copy(
                rows_v.at[pl.ds(j * B, B)],
                out_hbm.at[:, pl.ds((t0 + j) * D, D)],
                wsem,
            )
        for j in range(PPW):
            pltpu.make_async_copy(
                rows_v.at[pl.ds(j * B, B)],
                out_hbm.at[:, pl.ds((t0 + j) * D, D)],
                wsem,
            ).wait()


def kernel(sequence_embedding, tokens):
    table = sequence_embedding.reshape(B * V, D)
    return _sc_gather(table, tokens)
